# baseline (speedup 1.0000x reference)
import functools

import jax
import jax.numpy as jnp
from jax import lax
from jax.experimental import pallas as pl
from jax.experimental.pallas import tpu as pltpu
from jax.experimental.pallas import tpu_sc as plsc

N = 10000
E = 320000
D = 128
R_ET = 8
R_EMB = 4
R = R_ET + R_EMB

NC = 2
NS = 16
NW = NC * NS
EPW = E // NW
BV = 16
NB = EPW // BV
SBB = 5
SB = SBB * BV
NSB = NB // SBB
WW = 5 * 16
NZSUB = 10
RPS = N // NZSUB


def _prep_body(w12_ref, wfc_ref, wel_ref, weml_ref, b2_ref,
               wc_ref, wcl_ref, bias_ref):
    wf1 = wfc_ref[0:D, :]
    wf2 = wfc_ref[D:2 * D, :]
    for k in range(R):
        wf = wf1 if k < R_ET else wf2
        wc_ref[k] = jnp.dot(w12_ref[k], wf, preferred_element_type=jnp.float32)
    wcl_ref[...] = (jnp.dot(wel_ref[...], wf1, preferred_element_type=jnp.float32)
                    + jnp.dot(weml_ref[...], wf2, preferred_element_type=jnp.float32))
    bias_ref[...] = (jnp.dot(b2_ref[0:1, :], wf1, preferred_element_type=jnp.float32)
                     + jnp.dot(b2_ref[1:2, :], wf2, preferred_element_type=jnp.float32))


_prep_call = pl.pallas_call(
    _prep_body,
    out_shape=(
        jax.ShapeDtypeStruct((R, D, D), jnp.float32),
        jax.ShapeDtypeStruct((D, D), jnp.float32),
        jax.ShapeDtypeStruct((1, D), jnp.float32),
    ),
)


BN = 1000


def _dense_body(nf_ref, wc_ref, h_ref):
    h_ref[...] = jnp.dot(nf_ref[...], wc_ref[0],
                         preferred_element_type=jnp.float32)


_dense_call = pl.pallas_call(
    _dense_body,
    grid=(R, N // BN),
    in_specs=[
        pl.BlockSpec((BN, D), lambda k, i: (i, 0)),
        pl.BlockSpec((1, D, D), lambda k, i: (k, 0, 0)),
    ],
    out_specs=pl.BlockSpec((BN, D), lambda k, i: (i, k)),
    out_shape=jax.ShapeDtypeStruct((N, R * D), jnp.float32),
)


_mesh = plsc.VectorSubcoreMesh(core_axis_name="c", subcore_axis_name="s")


@functools.partial(
    pl.kernel,
    out_type=jax.ShapeDtypeStruct((NC, N, D), jnp.float32),
    mesh=_mesh,
    scratch_types=[
        pltpu.VMEM((2 * SB,), jnp.int32),
        pltpu.VMEM((2 * SB,), jnp.int32),
        pltpu.VMEM((2 * SB,), jnp.int32),
        pltpu.VMEM((2 * SB * WW,), jnp.float32),
        pltpu.VMEM((2, BV, D), jnp.float32),
        pltpu.VMEM((2, BV, 4 * D), jnp.float32),
        pltpu.VMEM((2, BV, D), jnp.float32),
        pltpu.VMEM_SHARED((N, D), jnp.float32),
        pltpu.SemaphoreType.DMA,
        pltpu.SemaphoreType.DMA((2,)),
        pltpu.SemaphoreType.DMA((2,)),
    ],
)
def _edge_call(tab_et, tab_emb, iet_hbm, iem_hbm, idst_hbm, wts_hbm, zeros_hbm,
               outp, m_iet, m_iem, m_idst, m_wts, ret_v, rem_v, msg_v, acc,
               sem_m, sem_g, sem_s):
    c = lax.axis_index("c")
    s = lax.axis_index("s")
    wid = c * NS + s

    @pl.when(s < NZSUB)
    def _zero():
        pltpu.sync_copy(zeros_hbm.at[pl.ds(s * RPS, RPS)],
                        acc.at[pl.ds(s * RPS, RPS)])
    plsc.subcore_barrier()

    def issue_meta(sb, slot):
        eb = wid * EPW + sb * SB
        pltpu.async_copy(iet_hbm.at[pl.ds(eb, SB)],
                         m_iet.at[pl.ds(slot * SB, SB)], sem_m)
        pltpu.async_copy(iem_hbm.at[pl.ds(eb, SB)],
                         m_iem.at[pl.ds(slot * SB, SB)], sem_m)
        pltpu.async_copy(idst_hbm.at[pl.ds(eb, SB)],
                         m_idst.at[pl.ds(slot * SB, SB)], sem_m)
        pltpu.async_copy(wts_hbm.at[pl.ds(eb * WW, SB * WW)],
                         m_wts.at[pl.ds(slot * SB * WW, SB * WW)], sem_m)

    def drain_meta(slot):
        pltpu.make_async_copy(iet_hbm.at[pl.ds(0, SB)],
                              m_iet.at[pl.ds(slot * SB, SB)], sem_m).wait()
        pltpu.make_async_copy(iem_hbm.at[pl.ds(0, SB)],
                              m_iem.at[pl.ds(slot * SB, SB)], sem_m).wait()
        pltpu.make_async_copy(idst_hbm.at[pl.ds(0, SB)],
                              m_idst.at[pl.ds(slot * SB, SB)], sem_m).wait()
        pltpu.make_async_copy(wts_hbm.at[pl.ds(0, SB * WW)],
                              m_wts.at[pl.ds(slot * SB * WW, SB * WW)],
                              sem_m).wait()

    def issue_gather(g):
        slot = lax.rem(g, 2)
        off = lax.rem(g // SBB, 2) * SB + lax.rem(g, SBB) * BV
        iet_vec = m_iet[pl.ds(off, BV)]
        iem_vec = m_iem[pl.ds(off, BV)]
        pltpu.async_copy(tab_et.at[iet_vec], ret_v.at[slot], sem_g.at[slot])
        pltpu.async_copy(tab_emb.at[iem_vec], rem_v.at[slot], sem_g.at[slot])

    def drain_gather(slot):
        pltpu.make_async_copy(tab_et.at[pl.ds(0, BV)], ret_v.at[slot],
                              sem_g.at[slot]).wait()
        pltpu.make_async_copy(tab_emb.at[pl.ds(0, BV)], rem_v.at[slot],
                              sem_g.at[slot]).wait()

    def drain_scatter(slot):
        pltpu.make_async_copy(msg_v.at[slot], acc.at[pl.ds(0, BV)],
                              sem_s.at[slot]).wait()

    issue_meta(0, 0)
    drain_meta(0)
    issue_gather(0)

    def batch_body(g, carry):
        nxt = g + 1
        slot = lax.rem(g, 2)
        sb_slot = lax.rem(g // SBB, 2)
        j = lax.rem(g, SBB)
        woff = sb_slot * (SB * WW) + j * (BV * WW)

        @pl.when(jnp.logical_and(nxt < NB, lax.rem(nxt, SBB) == 0))
        def _meta_arrived():
            drain_meta(lax.rem(nxt // SBB, 2))

        @pl.when(nxt < NB)
        def _prefetch():
            issue_gather(nxt)

        drain_gather(slot)

        @pl.when(g >= 2)
        def _msg_free():
            drain_scatter(slot)

        for i in range(BV):
            base = woff + i * WW
            w0 = m_wts[pl.ds(base, 16)]
            w1 = m_wts[pl.ds(base + 16, 16)]
            w2 = m_wts[pl.ds(base + 32, 16)]
            w3 = m_wts[pl.ds(base + 48, 16)]
            w4 = m_wts[pl.ds(base + 64, 16)]
            for dch in range(D // 16):
                sl = pl.ds(dch * 16, 16)
                v = ret_v[slot, i, sl] * w0
                v = v + rem_v[slot, i, pl.ds(0 * D + dch * 16, 16)] * w1
                v = v + rem_v[slot, i, pl.ds(1 * D + dch * 16, 16)] * w2
                v = v + rem_v[slot, i, pl.ds(2 * D + dch * 16, 16)] * w3
                v = v + rem_v[slot, i, pl.ds(3 * D + dch * 16, 16)] * w4
                msg_v[slot, i, sl] = v
        idst_vec = m_idst[pl.ds(sb_slot * SB + j * BV, BV)]
        pltpu.async_copy(msg_v.at[slot], acc.at[idst_vec],
                         sem_s.at[slot], add=True)

        @pl.when(jnp.logical_and(lax.rem(g, SBB) == 1, g // SBB + 1 < NSB))
        def _meta_prefetch():
            issue_meta(g // SBB + 1, lax.rem(g // SBB + 1, 2))

        return carry

    lax.fori_loop(0, NB, batch_body, 0)
    drain_scatter(0)
    drain_scatter(1)
    plsc.subcore_barrier()

    @pl.when(s < NZSUB)
    def _writeback():
        pltpu.sync_copy(acc.at[pl.ds(s * RPS, RPS)],
                        outp.at[c, pl.ds(s * RPS, RPS)])


def _final_body(p0_ref, p1_ref, nf_ref, wcl_ref, bias_ref, out_ref):
    out_ref[...] = (p0_ref[...] + p1_ref[...] + bias_ref[...]
                    + jnp.dot(nf_ref[...], wcl_ref[...],
                              preferred_element_type=jnp.float32))


_final_call = pl.pallas_call(
    _final_body,
    grid=(N // BN,),
    in_specs=[
        pl.BlockSpec((BN, D), lambda i: (i, 0)),
        pl.BlockSpec((BN, D), lambda i: (i, 0)),
        pl.BlockSpec((BN, D), lambda i: (i, 0)),
        pl.BlockSpec((D, D), lambda i: (0, 0)),
        pl.BlockSpec((1, D), lambda i: (0, 0)),
    ],
    out_specs=pl.BlockSpec((BN, D), lambda i: (i, 0)),
    out_shape=jax.ShapeDtypeStruct((N, D), jnp.float32),
)


def kernel(nfeats, edge_index, etypes, mask, edge_mask, efeats,
           W_et, W_et_loop, b_et, W_emb, W_emb_loop, b_emb, W_fc):
    src = edge_index[0]
    dst = edge_index[1]

    w12 = jnp.concatenate([W_et, W_emb], axis=0)
    b2 = jnp.stack([b_et, b_emb])
    wc_all, wcl, bias = _prep_call(w12, W_fc, W_et_loop, W_emb_loop, b2)

    h_all = _dense_call(nfeats, wc_all)
    tab_et = h_all.reshape(N * R, D)
    tab_emb = h_all.reshape(N * 3, 4 * D)

    iet = src * R + etypes
    iem = src * 3 + 2
    m = (edge_mask == 1.0).astype(jnp.float32)
    wts5 = jnp.stack([mask[:, 0],
                      edge_mask + efeats[:, 0] * m,
                      efeats[:, 1] * m,
                      efeats[:, 2] * m,
                      efeats[:, 3] * m], axis=1)
    wts = jnp.broadcast_to(wts5[:, :, None], (E, 5, 16)).reshape(-1)
    zeros = jnp.zeros((N, D), jnp.float32)

    outp = _edge_call(tab_et, tab_emb, iet, iem, dst, wts, zeros)
    return _final_call(outp[0], outp[1], nfeats, wcl, bias)


# speedup vs baseline: 1.0147x; 1.0147x over previous
import functools

import jax
import jax.numpy as jnp
from jax import lax
from jax.experimental import pallas as pl
from jax.experimental.pallas import tpu as pltpu
from jax.experimental.pallas import tpu_sc as plsc

N = 10000
E = 320000
D = 128
R_ET = 8
R_EMB = 4
R = R_ET + R_EMB

NC = 2
NS = 16
NW = NC * NS
EPW = E // NW
BV = 16
NB = EPW // BV
SBB = 5
SB = SBB * BV
NSB = NB // SBB
WW = 5 * 16
NZSUB = 10
RPS = N // NZSUB


def _prep_body(w12_ref, wfc_ref, wel_ref, weml_ref, b2_ref,
               wc_ref, wcl_ref, bias_ref):
    wf1 = wfc_ref[0:D, :]
    wf2 = wfc_ref[D:2 * D, :]
    for k in range(R):
        wf = wf1 if k < R_ET else wf2
        wc_ref[k] = jnp.dot(w12_ref[k], wf, preferred_element_type=jnp.float32)
    wcl_ref[...] = (jnp.dot(wel_ref[...], wf1, preferred_element_type=jnp.float32)
                    + jnp.dot(weml_ref[...], wf2, preferred_element_type=jnp.float32))
    bias_ref[...] = (jnp.dot(b2_ref[0:1, :], wf1, preferred_element_type=jnp.float32)
                     + jnp.dot(b2_ref[1:2, :], wf2, preferred_element_type=jnp.float32))


_prep_call = pl.pallas_call(
    _prep_body,
    out_shape=(
        jax.ShapeDtypeStruct((R, D, D), jnp.float32),
        jax.ShapeDtypeStruct((D, D), jnp.float32),
        jax.ShapeDtypeStruct((1, D), jnp.float32),
    ),
)


BN = 1000


def _dense_body(nf_ref, wc_ref, h_ref):
    h_ref[...] = jnp.dot(nf_ref[...], wc_ref[0],
                         preferred_element_type=jnp.float32)


_dense_call = pl.pallas_call(
    _dense_body,
    grid=(R, N // BN),
    in_specs=[
        pl.BlockSpec((BN, D), lambda k, i: (i, 0)),
        pl.BlockSpec((1, D, D), lambda k, i: (k, 0, 0)),
    ],
    out_specs=pl.BlockSpec((BN, D), lambda k, i: (i, k)),
    out_shape=jax.ShapeDtypeStruct((N, R * D), jnp.float32),
)


_mesh = plsc.VectorSubcoreMesh(core_axis_name="c", subcore_axis_name="s")


@functools.partial(
    pl.kernel,
    out_type=jax.ShapeDtypeStruct((NC, N, D), jnp.float32),
    mesh=_mesh,
    scratch_types=[
        pltpu.VMEM((2 * SB,), jnp.int32),
        pltpu.VMEM((2 * SB,), jnp.int32),
        pltpu.VMEM((2 * SB,), jnp.int32),
        pltpu.VMEM((2 * SB * WW,), jnp.float32),
        pltpu.VMEM((2, BV, D), jnp.float32),
        pltpu.VMEM((2, BV, 4 * D), jnp.float32),
        pltpu.VMEM((2, BV, D), jnp.float32),
        pltpu.VMEM_SHARED((N, D), jnp.float32),
        pltpu.SemaphoreType.DMA,
        pltpu.SemaphoreType.DMA((2,)),
        pltpu.SemaphoreType.DMA((2,)),
    ],
)
def _edge_call(tab_et, tab_emb, iet_hbm, iem_hbm, idst_hbm, wts_hbm, zeros_hbm,
               outp, m_iet, m_iem, m_idst, m_wts, ret_v, rem_v, msg_v, acc,
               sem_m, sem_g, sem_s):
    c = lax.axis_index("c")
    s = lax.axis_index("s")
    wid = c * NS + s

    @pl.when(s < NZSUB)
    def _zero():
        pltpu.sync_copy(zeros_hbm.at[pl.ds(s * RPS, RPS)],
                        acc.at[pl.ds(s * RPS, RPS)])
    plsc.subcore_barrier()

    def issue_meta(sb, slot):
        eb = wid * EPW + sb * SB
        pltpu.async_copy(iet_hbm.at[pl.ds(eb, SB)],
                         m_iet.at[pl.ds(slot * SB, SB)], sem_m)
        pltpu.async_copy(iem_hbm.at[pl.ds(eb, SB)],
                         m_iem.at[pl.ds(slot * SB, SB)], sem_m)
        pltpu.async_copy(idst_hbm.at[pl.ds(eb, SB)],
                         m_idst.at[pl.ds(slot * SB, SB)], sem_m)
        pltpu.async_copy(wts_hbm.at[pl.ds(eb * WW, SB * WW)],
                         m_wts.at[pl.ds(slot * SB * WW, SB * WW)], sem_m)

    def drain_meta(slot):
        pltpu.make_async_copy(iet_hbm.at[pl.ds(0, SB)],
                              m_iet.at[pl.ds(slot * SB, SB)], sem_m).wait()
        pltpu.make_async_copy(iem_hbm.at[pl.ds(0, SB)],
                              m_iem.at[pl.ds(slot * SB, SB)], sem_m).wait()
        pltpu.make_async_copy(idst_hbm.at[pl.ds(0, SB)],
                              m_idst.at[pl.ds(slot * SB, SB)], sem_m).wait()
        pltpu.make_async_copy(wts_hbm.at[pl.ds(0, SB * WW)],
                              m_wts.at[pl.ds(slot * SB * WW, SB * WW)],
                              sem_m).wait()

    def issue_gather(g):
        slot = lax.rem(g, 2)
        off = lax.rem(g // SBB, 2) * SB + lax.rem(g, SBB) * BV
        iet_vec = m_iet[pl.ds(off, BV)]
        iem_vec = m_iem[pl.ds(off, BV)]
        pltpu.async_copy(tab_et.at[iet_vec], ret_v.at[slot], sem_g.at[slot])
        pltpu.async_copy(tab_emb.at[iem_vec], rem_v.at[slot], sem_g.at[slot])

    def drain_gather(slot):
        pltpu.make_async_copy(tab_et.at[pl.ds(0, BV)], ret_v.at[slot],
                              sem_g.at[slot]).wait()
        pltpu.make_async_copy(tab_emb.at[pl.ds(0, BV)], rem_v.at[slot],
                              sem_g.at[slot]).wait()

    def drain_scatter(slot):
        pltpu.make_async_copy(msg_v.at[slot], acc.at[pl.ds(0, BV)],
                              sem_s.at[slot]).wait()

    issue_meta(0, 0)
    drain_meta(0)
    issue_gather(0)

    def batch_body(g, carry):
        nxt = g + 1
        slot = lax.rem(g, 2)
        sb_slot = lax.rem(g // SBB, 2)
        j = lax.rem(g, SBB)
        woff = sb_slot * (SB * WW) + j * (BV * WW)

        @pl.when(jnp.logical_and(nxt < NB, lax.rem(nxt, SBB) == 0))
        def _meta_arrived():
            drain_meta(lax.rem(nxt // SBB, 2))

        @pl.when(nxt < NB)
        def _prefetch():
            issue_gather(nxt)

        drain_gather(slot)

        @pl.when(g >= 2)
        def _msg_free():
            drain_scatter(slot)

        def edge_body(i, carry2):
            base = woff + i * WW
            w0 = m_wts[pl.ds(base, 16)]
            w1 = m_wts[pl.ds(base + 16, 16)]
            w2 = m_wts[pl.ds(base + 32, 16)]
            w3 = m_wts[pl.ds(base + 48, 16)]
            w4 = m_wts[pl.ds(base + 64, 16)]
            for dch in range(D // 16):
                sl = pl.ds(dch * 16, 16)
                v = ret_v[slot, i, sl] * w0
                v = v + rem_v[slot, i, pl.ds(0 * D + dch * 16, 16)] * w1
                v = v + rem_v[slot, i, pl.ds(1 * D + dch * 16, 16)] * w2
                v = v + rem_v[slot, i, pl.ds(2 * D + dch * 16, 16)] * w3
                v = v + rem_v[slot, i, pl.ds(3 * D + dch * 16, 16)] * w4
                msg_v[slot, i, sl] = v
            return carry2

        lax.fori_loop(0, BV, edge_body, 0)
        idst_vec = m_idst[pl.ds(sb_slot * SB + j * BV, BV)]
        pltpu.async_copy(msg_v.at[slot], acc.at[idst_vec],
                         sem_s.at[slot], add=True)

        @pl.when(jnp.logical_and(lax.rem(g, SBB) == 1, g // SBB + 1 < NSB))
        def _meta_prefetch():
            issue_meta(g // SBB + 1, lax.rem(g // SBB + 1, 2))

        return carry

    lax.fori_loop(0, NB, batch_body, 0)
    drain_scatter(0)
    drain_scatter(1)
    plsc.subcore_barrier()

    @pl.when(s < NZSUB)
    def _writeback():
        pltpu.sync_copy(acc.at[pl.ds(s * RPS, RPS)],
                        outp.at[c, pl.ds(s * RPS, RPS)])


def _final_body(p0_ref, p1_ref, nf_ref, wcl_ref, bias_ref, out_ref):
    out_ref[...] = (p0_ref[...] + p1_ref[...] + bias_ref[...]
                    + jnp.dot(nf_ref[...], wcl_ref[...],
                              preferred_element_type=jnp.float32))


_final_call = pl.pallas_call(
    _final_body,
    grid=(N // BN,),
    in_specs=[
        pl.BlockSpec((BN, D), lambda i: (i, 0)),
        pl.BlockSpec((BN, D), lambda i: (i, 0)),
        pl.BlockSpec((BN, D), lambda i: (i, 0)),
        pl.BlockSpec((D, D), lambda i: (0, 0)),
        pl.BlockSpec((1, D), lambda i: (0, 0)),
    ],
    out_specs=pl.BlockSpec((BN, D), lambda i: (i, 0)),
    out_shape=jax.ShapeDtypeStruct((N, D), jnp.float32),
)


def kernel(nfeats, edge_index, etypes, mask, edge_mask, efeats,
           W_et, W_et_loop, b_et, W_emb, W_emb_loop, b_emb, W_fc):
    src = edge_index[0]
    dst = edge_index[1]

    w12 = jnp.concatenate([W_et, W_emb], axis=0)
    b2 = jnp.stack([b_et, b_emb])
    wc_all, wcl, bias = _prep_call(w12, W_fc, W_et_loop, W_emb_loop, b2)

    h_all = _dense_call(nfeats, wc_all)
    tab_et = h_all.reshape(N * R, D)
    tab_emb = h_all.reshape(N * 3, 4 * D)

    iet = src * R + etypes
    iem = src * 3 + 2
    m = (edge_mask == 1.0).astype(jnp.float32)
    wts5 = jnp.stack([mask[:, 0],
                      edge_mask + efeats[:, 0] * m,
                      efeats[:, 1] * m,
                      efeats[:, 2] * m,
                      efeats[:, 3] * m], axis=1)
    wts = jnp.broadcast_to(wts5[:, :, None], (E, 5, 16)).reshape(-1)
    zeros = jnp.zeros((N, D), jnp.float32)

    outp = _edge_call(tab_et, tab_emb, iet, iem, dst, wts, zeros)
    return _final_call(outp[0], outp[1], nfeats, wcl, bias)


# speedup vs baseline: 1.8275x; 1.8010x over previous
import functools

import jax
import jax.numpy as jnp
from jax import lax
from jax.experimental import pallas as pl
from jax.experimental.pallas import tpu as pltpu
from jax.experimental.pallas import tpu_sc as plsc

N = 10000
E = 320000
D = 128
R_ET = 8
R_EMB = 4
R = R_ET + R_EMB

NC = 2
NS = 16
NW = NC * NS
EPW = E // NW
BV = 16
NB = EPW // BV
SBB = 5
SB = SBB * BV
NSB = NB // SBB
WW = 5 * 16
NZSUB = 10
RPS = N // NZSUB


def _prep_body(w12_ref, wfc_ref, wel_ref, weml_ref, b2_ref,
               wc_ref, wcl_ref, bias_ref):
    wf1 = wfc_ref[0:D, :]
    wf2 = wfc_ref[D:2 * D, :]
    for k in range(R):
        wf = wf1 if k < R_ET else wf2
        wc_ref[k] = jnp.dot(w12_ref[k], wf, preferred_element_type=jnp.float32)
    wcl_ref[...] = (jnp.dot(wel_ref[...], wf1, preferred_element_type=jnp.float32)
                    + jnp.dot(weml_ref[...], wf2, preferred_element_type=jnp.float32))
    bias_ref[...] = (jnp.dot(b2_ref[0:1, :], wf1, preferred_element_type=jnp.float32)
                     + jnp.dot(b2_ref[1:2, :], wf2, preferred_element_type=jnp.float32))


_prep_call = pl.pallas_call(
    _prep_body,
    out_shape=(
        jax.ShapeDtypeStruct((R, D, D), jnp.float32),
        jax.ShapeDtypeStruct((D, D), jnp.float32),
        jax.ShapeDtypeStruct((1, D), jnp.float32),
    ),
)


BN = 1000


def _dense_body(nf_ref, wc_ref, h_ref):
    h_ref[...] = jnp.dot(nf_ref[...], wc_ref[0],
                         preferred_element_type=jnp.float32)


_dense_call = pl.pallas_call(
    _dense_body,
    grid=(R, N // BN),
    in_specs=[
        pl.BlockSpec((BN, D), lambda k, i: (i, 0)),
        pl.BlockSpec((1, D, D), lambda k, i: (k, 0, 0)),
    ],
    out_specs=pl.BlockSpec((BN, D), lambda k, i: (i, k)),
    out_shape=jax.ShapeDtypeStruct((N, R * D), jnp.float32),
)


_mesh = plsc.VectorSubcoreMesh(core_axis_name="c", subcore_axis_name="s")


@functools.partial(
    pl.kernel,
    out_type=jax.ShapeDtypeStruct((NC, N, D), jnp.float32),
    mesh=_mesh,
    scratch_types=[
        pltpu.VMEM((2 * SB,), jnp.int32),
        pltpu.VMEM((2 * SB,), jnp.int32),
        pltpu.VMEM((2 * SB,), jnp.int32),
        pltpu.VMEM((2 * (SB * 5 + 16),), jnp.float32),
        pltpu.VMEM((2, BV, D), jnp.float32),
        pltpu.VMEM((2, BV, 4 * D), jnp.float32),
        pltpu.VMEM((2, BV, D), jnp.float32),
        pltpu.VMEM_SHARED((N, D), jnp.float32),
        pltpu.SemaphoreType.DMA,
        pltpu.SemaphoreType.DMA((2,)),
        pltpu.SemaphoreType.DMA((2,)),
    ],
)
def _edge_call(tab_et, tab_emb, iet_hbm, iem_hbm, idst_hbm, wts_hbm, zeros_hbm,
               outp, m_iet, m_iem, m_idst, m_wts, ret_v, rem_v, msg_v, acc,
               sem_m, sem_g, sem_s):
    c = lax.axis_index("c")
    s = lax.axis_index("s")
    wid = c * NS + s

    @pl.when(s < NZSUB)
    def _zero():
        pltpu.sync_copy(zeros_hbm.at[pl.ds(s * RPS, RPS)],
                        acc.at[pl.ds(s * RPS, RPS)])
    plsc.subcore_barrier()

    def issue_meta(sb, slot):
        eb = wid * EPW + sb * SB
        pltpu.async_copy(iet_hbm.at[pl.ds(eb, SB)],
                         m_iet.at[pl.ds(slot * SB, SB)], sem_m)
        pltpu.async_copy(iem_hbm.at[pl.ds(eb, SB)],
                         m_iem.at[pl.ds(slot * SB, SB)], sem_m)
        pltpu.async_copy(idst_hbm.at[pl.ds(eb, SB)],
                         m_idst.at[pl.ds(slot * SB, SB)], sem_m)
        pltpu.async_copy(wts_hbm.at[pl.ds(eb * 5, SB * 5)],
                         m_wts.at[pl.ds(slot * (SB * 5 + 16), SB * 5)], sem_m)

    def drain_meta(slot):
        pltpu.make_async_copy(iet_hbm.at[pl.ds(0, SB)],
                              m_iet.at[pl.ds(slot * SB, SB)], sem_m).wait()
        pltpu.make_async_copy(iem_hbm.at[pl.ds(0, SB)],
                              m_iem.at[pl.ds(slot * SB, SB)], sem_m).wait()
        pltpu.make_async_copy(idst_hbm.at[pl.ds(0, SB)],
                              m_idst.at[pl.ds(slot * SB, SB)], sem_m).wait()
        pltpu.make_async_copy(wts_hbm.at[pl.ds(0, SB * 5)],
                              m_wts.at[pl.ds(slot * (SB * 5 + 16), SB * 5)],
                              sem_m).wait()

    def issue_gather(g):
        slot = lax.rem(g, 2)
        off = lax.rem(g // SBB, 2) * SB + lax.rem(g, SBB) * BV
        iet_vec = m_iet[pl.ds(off, BV)]
        iem_vec = m_iem[pl.ds(off, BV)]
        pltpu.async_copy(tab_et.at[iet_vec], ret_v.at[slot], sem_g.at[slot])
        pltpu.async_copy(tab_emb.at[iem_vec], rem_v.at[slot], sem_g.at[slot])

    def drain_gather(slot):
        pltpu.make_async_copy(tab_et.at[pl.ds(0, BV)], ret_v.at[slot],
                              sem_g.at[slot]).wait()
        pltpu.make_async_copy(tab_emb.at[pl.ds(0, BV)], rem_v.at[slot],
                              sem_g.at[slot]).wait()

    def drain_scatter(slot):
        pltpu.make_async_copy(msg_v.at[slot], acc.at[pl.ds(0, BV)],
                              sem_s.at[slot]).wait()

    issue_meta(0, 0)
    drain_meta(0)
    issue_gather(0)

    def batch_body(g, carry):
        nxt = g + 1
        slot = lax.rem(g, 2)
        sb_slot = lax.rem(g // SBB, 2)
        j = lax.rem(g, SBB)
        woff = sb_slot * (SB * 5 + 16) + j * (BV * 5)

        @pl.when(jnp.logical_and(nxt < NB, lax.rem(nxt, SBB) == 0))
        def _meta_arrived():
            drain_meta(lax.rem(nxt // SBB, 2))

        @pl.when(nxt < NB)
        def _prefetch():
            issue_gather(nxt)

        drain_gather(slot)

        @pl.when(g >= 2)
        def _msg_free():
            drain_scatter(slot)

        def edge_body(i, carry2):
            wv = m_wts[pl.ds(woff + i * 5, 16)]
            w0 = wv[0]
            w1 = wv[1]
            w2 = wv[2]
            w3 = wv[3]
            w4 = wv[4]
            for dch in range(D // 16):
                sl = pl.ds(dch * 16, 16)
                v = ret_v[slot, i, sl] * w0
                v = v + rem_v[slot, i, pl.ds(0 * D + dch * 16, 16)] * w1
                v = v + rem_v[slot, i, pl.ds(1 * D + dch * 16, 16)] * w2
                v = v + rem_v[slot, i, pl.ds(2 * D + dch * 16, 16)] * w3
                v = v + rem_v[slot, i, pl.ds(3 * D + dch * 16, 16)] * w4
                msg_v[slot, i, sl] = v
            return carry2

        lax.fori_loop(0, BV, edge_body, 0)
        idst_vec = m_idst[pl.ds(sb_slot * SB + j * BV, BV)]
        pltpu.async_copy(msg_v.at[slot], acc.at[idst_vec],
                         sem_s.at[slot], add=True)

        @pl.when(jnp.logical_and(lax.rem(g, SBB) == 1, g // SBB + 1 < NSB))
        def _meta_prefetch():
            issue_meta(g // SBB + 1, lax.rem(g // SBB + 1, 2))

        return carry

    lax.fori_loop(0, NB, batch_body, 0)
    drain_scatter(0)
    drain_scatter(1)
    plsc.subcore_barrier()

    @pl.when(s < NZSUB)
    def _writeback():
        pltpu.sync_copy(acc.at[pl.ds(s * RPS, RPS)],
                        outp.at[c, pl.ds(s * RPS, RPS)])


def _final_body(p0_ref, p1_ref, nf_ref, wcl_ref, bias_ref, out_ref):
    out_ref[...] = (p0_ref[...] + p1_ref[...] + bias_ref[...]
                    + jnp.dot(nf_ref[...], wcl_ref[...],
                              preferred_element_type=jnp.float32))


_final_call = pl.pallas_call(
    _final_body,
    grid=(N // BN,),
    in_specs=[
        pl.BlockSpec((BN, D), lambda i: (i, 0)),
        pl.BlockSpec((BN, D), lambda i: (i, 0)),
        pl.BlockSpec((BN, D), lambda i: (i, 0)),
        pl.BlockSpec((D, D), lambda i: (0, 0)),
        pl.BlockSpec((1, D), lambda i: (0, 0)),
    ],
    out_specs=pl.BlockSpec((BN, D), lambda i: (i, 0)),
    out_shape=jax.ShapeDtypeStruct((N, D), jnp.float32),
)


def kernel(nfeats, edge_index, etypes, mask, edge_mask, efeats,
           W_et, W_et_loop, b_et, W_emb, W_emb_loop, b_emb, W_fc):
    src = edge_index[0]
    dst = edge_index[1]

    w12 = jnp.concatenate([W_et, W_emb], axis=0)
    b2 = jnp.stack([b_et, b_emb])
    wc_all, wcl, bias = _prep_call(w12, W_fc, W_et_loop, W_emb_loop, b2)

    h_all = _dense_call(nfeats, wc_all)
    tab_et = h_all.reshape(N * R, D)
    tab_emb = h_all.reshape(N * 3, 4 * D)

    iet = src * R + etypes
    iem = src * 3 + 2
    m = (edge_mask == 1.0).astype(jnp.float32)
    wts = jnp.stack([mask[:, 0],
                     edge_mask + efeats[:, 0] * m,
                     efeats[:, 1] * m,
                     efeats[:, 2] * m,
                     efeats[:, 3] * m], axis=1).reshape(-1)
    zeros = jnp.zeros((N, D), jnp.float32)

    outp = _edge_call(tab_et, tab_emb, iet, iem, dst, wts, zeros)
    return _final_call(outp[0], outp[1], nfeats, wcl, bias)
